# baseline (speedup 1.0000x reference)
import functools

import jax
import jax.numpy as jnp
from jax import lax
from jax.experimental import pallas as pl
from jax.experimental.pallas import tpu as pltpu
from jax.experimental.pallas import tpu_sc as plsc

NUM_EXPERTS = 8
TOP_K = 2
HIDDEN = 1024
EXPERT_DIM = 1024
ALPHA = 1.702
LIMIT = 7.0

BLK = 256
NBLK = 2048 * TOP_K // BLK + NUM_EXPERTS
P = NBLK * BLK

NC = 2
NS = 16
NW = NC * NS


def _gather_body(x_hbm, idx_hbm, out_hbm, idx_v, rows_v, sem):
    wid = lax.axis_index("s") * NC + lax.axis_index("c")
    rows_per_w = P // NW
    ch = 96
    for c in range(rows_per_w // ch):
        base = wid * rows_per_w + c * ch
        pltpu.sync_copy(idx_hbm.at[pl.ds(base, ch)], idx_v)
        pltpu.async_copy(x_hbm.at[idx_v], rows_v, sem).wait()
        pltpu.sync_copy(rows_v, out_hbm.at[pl.ds(base, ch)])


def _sc_gather(x, row_ids):
    mesh = plsc.VectorSubcoreMesh(core_axis_name="c", subcore_axis_name="s")
    f = functools.partial(
        pl.kernel,
        out_type=jax.ShapeDtypeStruct((P, HIDDEN), jnp.float32),
        mesh=mesh,
        scratch_types=[
            pltpu.VMEM((96,), jnp.int32),
            pltpu.VMEM((96, HIDDEN), jnp.float32),
            pltpu.SemaphoreType.DMA,
        ],
    )(_gather_body)
    return f(x, row_ids)


_CCH = 32


def _combine_body(ys_hbm, pos_hbm, out_hbm, idx0_v, idx1_v, a_v, b_v,
                  sem0, sem1):
    wid = lax.axis_index("s") * NC + lax.axis_index("c")
    tok_per_w = 2048 // NW
    for c in range(tok_per_w // _CCH):
        base = wid * tok_per_w + c * _CCH
        pltpu.sync_copy(pos_hbm.at[0, pl.ds(base, _CCH)], idx0_v)
        pltpu.sync_copy(pos_hbm.at[1, pl.ds(base, _CCH)], idx1_v)
        cp0 = pltpu.async_copy(ys_hbm.at[idx0_v], a_v, sem0)
        cp1 = pltpu.async_copy(ys_hbm.at[idx1_v], b_v, sem1)
        cp0.wait()
        cp1.wait()

        def _row(i, _):
            for j in range(HIDDEN // 16):
                s = pl.ds(j * 16, 16)
                a_v[i, s] = a_v[i, s] + b_v[i, s]
            return _

        lax.fori_loop(0, _CCH, _row, 0)
        pltpu.sync_copy(a_v, out_hbm.at[pl.ds(base, _CCH)])


def _sc_combine(ys, pos):
    mesh = plsc.VectorSubcoreMesh(core_axis_name="c", subcore_axis_name="s")
    f = functools.partial(
        pl.kernel,
        out_type=jax.ShapeDtypeStruct((2048, HIDDEN), jnp.float32),
        mesh=mesh,
        scratch_types=[
            pltpu.VMEM((_CCH,), jnp.int32),
            pltpu.VMEM((_CCH,), jnp.int32),
            pltpu.VMEM((_CCH, HIDDEN), jnp.float32),
            pltpu.VMEM((_CCH, HIDDEN), jnp.float32),
            pltpu.SemaphoreType.DMA,
            pltpu.SemaphoreType.DMA,
        ],
    )(_combine_body)
    return f(ys, pos)


def _ffn_body(be_ref, xs_ref, wgu_ref, wd_ref, bgu_ref, bd_ref, wp_ref,
              sel_ref, ys_ref):
    xb = xs_ref[...].astype(jnp.bfloat16)
    gu = jnp.dot(xb, wgu_ref[0], preferred_element_type=jnp.float32)
    gu = gu + bgu_ref[0, 0]
    gate = jnp.minimum(gu, LIMIT)
    glu = gate * jax.nn.sigmoid(gate * ALPHA)
    up1 = jnp.clip(gu, -LIMIT, LIMIT) + 1.0
    up1 = pltpu.roll(up1, 2 * EXPERT_DIM - 1, 1)
    h = (glu * up1).astype(jnp.bfloat16)
    hc = jnp.dot(h, sel_ref[...], preferred_element_type=jnp.float32)
    hc = hc.astype(jnp.bfloat16)
    y = jnp.dot(hc, wd_ref[0], preferred_element_type=jnp.float32) + bd_ref[0, 0]
    ys_ref[...] = y * wp_ref[0, 0][:, None]


def _tc_ffn(xs, wgu, wd, bgu, bd, wpad, block_expert, sel):
    grid_spec = pltpu.PrefetchScalarGridSpec(
        num_scalar_prefetch=1,
        grid=(NBLK,),
        in_specs=[
            pl.BlockSpec((BLK, HIDDEN), lambda b, be: (b, 0)),
            pl.BlockSpec((1, HIDDEN, 2 * EXPERT_DIM), lambda b, be: (be[b], 0, 0)),
            pl.BlockSpec((1, EXPERT_DIM, HIDDEN), lambda b, be: (be[b], 0, 0)),
            pl.BlockSpec((1, 1, 2 * EXPERT_DIM), lambda b, be: (be[b], 0, 0)),
            pl.BlockSpec((1, 1, HIDDEN), lambda b, be: (be[b], 0, 0)),
            pl.BlockSpec((1, 1, BLK), lambda b, be: (b, 0, 0)),
            pl.BlockSpec((2 * EXPERT_DIM, EXPERT_DIM), lambda b, be: (0, 0)),
        ],
        out_specs=pl.BlockSpec((BLK, HIDDEN), lambda b, be: (b, 0)),
    )
    return pl.pallas_call(
        _ffn_body,
        grid_spec=grid_spec,
        out_shape=jax.ShapeDtypeStruct((P, HIDDEN), jnp.float32),
    )(block_expert, xs, wgu, wd, bgu, bd, wpad, sel)


def kernel(hidden_states, router_indices, routing_weights, gate_up_proj,
           gate_up_proj_bias, down_proj, down_proj_bias):
    batch, seq, hidden = hidden_states.shape
    T = batch * seq
    R = T * TOP_K
    x = hidden_states.reshape(T, hidden).astype(jnp.float32)

    ids = router_indices.reshape(R).astype(jnp.int32)
    oh = (ids[:, None] == jnp.arange(NUM_EXPERTS, dtype=jnp.int32)).astype(jnp.int32)
    counts = jnp.sum(oh, axis=0)
    rank = jnp.take_along_axis(jnp.cumsum(oh, axis=0) - oh,
                               ids[:, None], axis=1)[:, 0]
    nblk_e = (counts + BLK - 1) // BLK
    cum_nblk = jnp.cumsum(nblk_e)
    blk_start = cum_nblk - nblk_e
    dest = blk_start[ids] * BLK + rank
    tok = jnp.arange(R, dtype=jnp.int32) // TOP_K
    row_ids = jnp.zeros((P,), jnp.int32).at[dest].set(tok)
    wrow = jnp.take_along_axis(routing_weights, ids.reshape(T, TOP_K),
                               axis=1).reshape(R)
    wpad = jnp.zeros((P,), jnp.float32).at[dest].set(wrow)
    bid = jnp.arange(NBLK, dtype=jnp.int32)
    block_expert = jnp.minimum(
        jnp.sum((bid[:, None] >= cum_nblk[None, :]).astype(jnp.int32), axis=1),
        NUM_EXPERTS - 1)
    pos = dest.reshape(T, TOP_K).T

    wgu = gate_up_proj.astype(jnp.bfloat16)
    bgu = gate_up_proj_bias.reshape(NUM_EXPERTS, 1, 2 * EXPERT_DIM)
    wd = down_proj.astype(jnp.bfloat16)
    bd = down_proj_bias.reshape(NUM_EXPERTS, 1, HIDDEN)
    r2 = jax.lax.broadcasted_iota(jnp.int32, (2 * EXPERT_DIM, EXPERT_DIM), 0)
    c2 = jax.lax.broadcasted_iota(jnp.int32, (2 * EXPERT_DIM, EXPERT_DIM), 1)
    sel = (r2 == 2 * c2).astype(jnp.bfloat16)

    xs = _sc_gather(x, row_ids)
    ys = _tc_ffn(xs, wgu, wd, bgu, bd,
                 wpad.reshape(NBLK, 1, BLK), block_expert, sel)
    out = _sc_combine(ys, pos)
    return out.astype(hidden_states.dtype).reshape(batch, seq, hidden)


# speedup vs baseline: 1.6270x; 1.6270x over previous
import functools

import jax
import jax.numpy as jnp
from jax import lax
from jax.experimental import pallas as pl
from jax.experimental.pallas import tpu as pltpu
from jax.experimental.pallas import tpu_sc as plsc

NUM_EXPERTS = 8
TOP_K = 2
HIDDEN = 1024
EXPERT_DIM = 1024
ALPHA = 1.702
LIMIT = 7.0

BLK = 256
NBLK = 2048 * TOP_K // BLK + NUM_EXPERTS
P = NBLK * BLK

NC = 2
NS = 16
NW = NC * NS


def _dispatch_body(x_hbm, pos_hbm, out_hbm, idx0_v, idx1_v, buf_v, sem):
    wid = lax.axis_index("s") * NC + lax.axis_index("c")
    tok_per_w = 2048 // NW
    base = wid * tok_per_w
    pltpu.sync_copy(pos_hbm.at[0, pl.ds(base, tok_per_w)], idx0_v)
    pltpu.sync_copy(pos_hbm.at[1, pl.ds(base, tok_per_w)], idx1_v)
    pltpu.sync_copy(x_hbm.at[pl.ds(base, tok_per_w)], buf_v)
    pltpu.async_copy(buf_v, out_hbm.at[idx0_v], sem).wait()
    pltpu.async_copy(buf_v, out_hbm.at[idx1_v], sem).wait()


def _sc_dispatch(x, pos):
    mesh = plsc.VectorSubcoreMesh(core_axis_name="c", subcore_axis_name="s")
    f = functools.partial(
        pl.kernel,
        out_type=jax.ShapeDtypeStruct((P, HIDDEN), jnp.float32),
        mesh=mesh,
        scratch_types=[
            pltpu.VMEM((64,), jnp.int32),
            pltpu.VMEM((64,), jnp.int32),
            pltpu.VMEM((64, HIDDEN), jnp.float32),
            pltpu.SemaphoreType.DMA,
        ],
    )(_dispatch_body)
    return f(x, pos)


_CCH = 32


def _combine_body(ys_hbm, pos_hbm, wb_hbm, out_hbm, idx0_v, idx1_v, a_v, b_v,
                  w0_v, w1_v, sem0, sem1):
    wid = lax.axis_index("s") * NC + lax.axis_index("c")
    tok_per_w = 2048 // NW
    for c in range(tok_per_w // _CCH):
        base = wid * tok_per_w + c * _CCH
        pltpu.sync_copy(pos_hbm.at[0, pl.ds(base, _CCH)], idx0_v)
        pltpu.sync_copy(pos_hbm.at[1, pl.ds(base, _CCH)], idx1_v)
        pltpu.sync_copy(wb_hbm.at[0, pl.ds(base, _CCH)], w0_v)
        pltpu.sync_copy(wb_hbm.at[1, pl.ds(base, _CCH)], w1_v)
        cp0 = pltpu.async_copy(ys_hbm.at[idx0_v], a_v, sem0)
        cp1 = pltpu.async_copy(ys_hbm.at[idx1_v], b_v, sem1)
        cp0.wait()
        cp1.wait()

        def _row(i, _):
            w0 = w0_v[i]
            w1 = w1_v[i]
            for j in range(HIDDEN // 16):
                s = pl.ds(j * 16, 16)
                a_v[i, s] = a_v[i, s] * w0 + b_v[i, s] * w1
            return _

        lax.fori_loop(0, _CCH, _row, 0)
        pltpu.sync_copy(a_v, out_hbm.at[pl.ds(base, _CCH)])


def _sc_combine(ys, pos, wb):
    mesh = plsc.VectorSubcoreMesh(core_axis_name="c", subcore_axis_name="s")
    f = functools.partial(
        pl.kernel,
        out_type=jax.ShapeDtypeStruct((2048, HIDDEN), jnp.float32),
        mesh=mesh,
        scratch_types=[
            pltpu.VMEM((_CCH,), jnp.int32),
            pltpu.VMEM((_CCH,), jnp.int32),
            pltpu.VMEM((_CCH, HIDDEN), jnp.float32),
            pltpu.VMEM((_CCH, HIDDEN), jnp.float32),
            pltpu.VMEM((_CCH, 16), jnp.float32),
            pltpu.VMEM((_CCH, 16), jnp.float32),
            pltpu.SemaphoreType.DMA,
            pltpu.SemaphoreType.DMA,
        ],
    )(_combine_body)
    return f(ys, pos, wb)


def _ffn_body(be_ref, xs_ref, wgu_ref, wd_ref, bgu_ref, bd_ref,
              sel_ref, ys_ref):
    xb = xs_ref[...].astype(jnp.bfloat16)
    gu = jnp.dot(xb, wgu_ref[0], preferred_element_type=jnp.float32)
    gu = gu + bgu_ref[0, 0]
    gate = jnp.minimum(gu, LIMIT)
    glu = gate * jax.nn.sigmoid(gate * ALPHA)
    up1 = jnp.clip(gu, -LIMIT, LIMIT) + 1.0
    up1 = pltpu.roll(up1, 2 * EXPERT_DIM - 1, 1)
    h = (glu * up1).astype(jnp.bfloat16)
    hc = jnp.dot(h, sel_ref[...], preferred_element_type=jnp.float32)
    hc = hc.astype(jnp.bfloat16)
    y = jnp.dot(hc, wd_ref[0], preferred_element_type=jnp.float32) + bd_ref[0, 0]
    ys_ref[...] = y


def _tc_ffn(xs, wgu, wd, bgu, bd, block_expert, sel):
    grid_spec = pltpu.PrefetchScalarGridSpec(
        num_scalar_prefetch=1,
        grid=(NBLK,),
        in_specs=[
            pl.BlockSpec((BLK, HIDDEN), lambda b, be: (b, 0)),
            pl.BlockSpec((1, HIDDEN, 2 * EXPERT_DIM), lambda b, be: (be[b], 0, 0)),
            pl.BlockSpec((1, EXPERT_DIM, HIDDEN), lambda b, be: (be[b], 0, 0)),
            pl.BlockSpec((1, 1, 2 * EXPERT_DIM), lambda b, be: (be[b], 0, 0)),
            pl.BlockSpec((1, 1, HIDDEN), lambda b, be: (be[b], 0, 0)),
            pl.BlockSpec((2 * EXPERT_DIM, EXPERT_DIM), lambda b, be: (0, 0)),
        ],
        out_specs=pl.BlockSpec((BLK, HIDDEN), lambda b, be: (b, 0)),
    )
    return pl.pallas_call(
        _ffn_body,
        grid_spec=grid_spec,
        out_shape=jax.ShapeDtypeStruct((P, HIDDEN), jnp.float32),
    )(block_expert, xs, wgu, wd, bgu, bd, sel)


def kernel(hidden_states, router_indices, routing_weights, gate_up_proj,
           gate_up_proj_bias, down_proj, down_proj_bias):
    batch, seq, hidden = hidden_states.shape
    T = batch * seq
    R = T * TOP_K
    x = hidden_states.reshape(T, hidden).astype(jnp.float32)

    ids = router_indices.reshape(R).astype(jnp.int32)
    oh = (ids[:, None] == jnp.arange(NUM_EXPERTS, dtype=jnp.int32)).astype(jnp.int32)
    counts = jnp.sum(oh, axis=0)
    rank = jnp.take_along_axis(jnp.cumsum(oh, axis=0) - oh,
                               ids[:, None], axis=1)[:, 0]
    nblk_e = (counts + BLK - 1) // BLK
    cum_nblk = jnp.cumsum(nblk_e)
    blk_start = cum_nblk - nblk_e
    dest = blk_start[ids] * BLK + rank
    bid = jnp.arange(NBLK, dtype=jnp.int32)
    block_expert = jnp.minimum(
        jnp.sum((bid[:, None] >= cum_nblk[None, :]).astype(jnp.int32), axis=1),
        NUM_EXPERTS - 1)
    pos = dest.reshape(T, TOP_K).T
    wrow = jnp.take_along_axis(routing_weights, ids.reshape(T, TOP_K),
                               axis=1)
    wb = jnp.broadcast_to(wrow.T[:, :, None], (TOP_K, T, 16))

    wgu = gate_up_proj.astype(jnp.bfloat16)
    bgu = gate_up_proj_bias.reshape(NUM_EXPERTS, 1, 2 * EXPERT_DIM)
    wd = down_proj.astype(jnp.bfloat16)
    bd = down_proj_bias.reshape(NUM_EXPERTS, 1, HIDDEN)
    r2 = jax.lax.broadcasted_iota(jnp.int32, (2 * EXPERT_DIM, EXPERT_DIM), 0)
    c2 = jax.lax.broadcasted_iota(jnp.int32, (2 * EXPERT_DIM, EXPERT_DIM), 1)
    sel = (r2 == 2 * c2).astype(jnp.bfloat16)

    xs = _sc_dispatch(x, pos)
    ys = _tc_ffn(xs, wgu, wd, bgu, bd, block_expert, sel)
    out = _sc_combine(ys, pos, wb)
    return out.astype(hidden_states.dtype).reshape(batch, seq, hidden)


# speedup vs baseline: 1.8485x; 1.1361x over previous
import functools

import jax
import jax.numpy as jnp
from jax import lax
from jax.experimental import pallas as pl
from jax.experimental.pallas import tpu as pltpu
from jax.experimental.pallas import tpu_sc as plsc

NUM_EXPERTS = 8
TOP_K = 2
HIDDEN = 1024
EXPERT_DIM = 1024
ALPHA = 1.702
LIMIT = 7.0

BLK = 256
NBLK = 2048 * TOP_K // BLK + NUM_EXPERTS
P = NBLK * BLK

NC = 2
NS = 16
NW = NC * NS


def _dispatch_body(x_hbm, pos_hbm, out_hbm, idx0_v, idx1_v, buf_v,
                   sem0, sem1):
    wid = lax.axis_index("s") * NC + lax.axis_index("c")
    tok_per_w = 2048 // NW
    base = wid * tok_per_w
    pltpu.sync_copy(pos_hbm.at[0, pl.ds(base, tok_per_w)], idx0_v)
    pltpu.sync_copy(pos_hbm.at[1, pl.ds(base, tok_per_w)], idx1_v)
    pltpu.sync_copy(x_hbm.at[pl.ds(base, tok_per_w)], buf_v)
    cp0 = pltpu.async_copy(buf_v, out_hbm.at[idx0_v], sem0)
    cp1 = pltpu.async_copy(buf_v, out_hbm.at[idx1_v], sem1)
    cp0.wait()
    cp1.wait()


def _sc_dispatch(x, pos):
    mesh = plsc.VectorSubcoreMesh(core_axis_name="c", subcore_axis_name="s")
    f = functools.partial(
        pl.kernel,
        out_type=jax.ShapeDtypeStruct((P, HIDDEN), jnp.float32),
        mesh=mesh,
        scratch_types=[
            pltpu.VMEM((64,), jnp.int32),
            pltpu.VMEM((64,), jnp.int32),
            pltpu.VMEM((64, HIDDEN), jnp.float32),
            pltpu.SemaphoreType.DMA,
            pltpu.SemaphoreType.DMA,
        ],
    )(_dispatch_body)
    return f(x, pos)


_CCH = 32


def _combine_body(ys_hbm, pos_hbm, wb_hbm, out_hbm, idx0_v, idx1_v, a_v, b_v,
                  w0_v, w1_v, sem0, sem1):
    wid = lax.axis_index("s") * NC + lax.axis_index("c")
    tok_per_w = 2048 // NW
    for c in range(tok_per_w // _CCH):
        base = wid * tok_per_w + c * _CCH
        pltpu.sync_copy(pos_hbm.at[0, pl.ds(base, _CCH)], idx0_v)
        pltpu.sync_copy(pos_hbm.at[1, pl.ds(base, _CCH)], idx1_v)
        pltpu.sync_copy(wb_hbm.at[0, pl.ds(base, _CCH)], w0_v)
        pltpu.sync_copy(wb_hbm.at[1, pl.ds(base, _CCH)], w1_v)
        cp0 = pltpu.async_copy(ys_hbm.at[idx0_v], a_v, sem0)
        cp1 = pltpu.async_copy(ys_hbm.at[idx1_v], b_v, sem1)
        cp0.wait()
        cp1.wait()

        def _row(i, _):
            w0 = w0_v[i]
            w1 = w1_v[i]
            for j in range(HIDDEN // 16):
                s = pl.ds(j * 16, 16)
                a_v[i, s] = a_v[i, s] * w0 + b_v[i, s] * w1
            return _

        lax.fori_loop(0, _CCH, _row, 0)
        pltpu.sync_copy(a_v, out_hbm.at[pl.ds(base, _CCH)])


def _sc_combine(ys, pos, wb):
    mesh = plsc.VectorSubcoreMesh(core_axis_name="c", subcore_axis_name="s")
    f = functools.partial(
        pl.kernel,
        out_type=jax.ShapeDtypeStruct((2048, HIDDEN), jnp.float32),
        mesh=mesh,
        scratch_types=[
            pltpu.VMEM((_CCH,), jnp.int32),
            pltpu.VMEM((_CCH,), jnp.int32),
            pltpu.VMEM((_CCH, HIDDEN), jnp.float32),
            pltpu.VMEM((_CCH, HIDDEN), jnp.float32),
            pltpu.VMEM((_CCH, 16), jnp.float32),
            pltpu.VMEM((_CCH, 16), jnp.float32),
            pltpu.SemaphoreType.DMA,
            pltpu.SemaphoreType.DMA,
        ],
    )(_combine_body)
    return f(ys, pos, wb)


def _ffn_body(be_ref, xs_ref, wgu_ref, wd_ref, bgu_ref, bd_ref,
              sel_ref, ys_ref, wgu_s, wd_s):
    b = pl.program_id(0)
    prev = be_ref[jnp.maximum(b - 1, 0)]
    refresh = jnp.logical_or(b == 0, be_ref[b] != prev)

    @pl.when(refresh)
    def _cast_weights():
        wgu_s[...] = wgu_ref[0].astype(jnp.bfloat16)
        wd_s[...] = wd_ref[0].astype(jnp.bfloat16)

    xb = xs_ref[...].astype(jnp.bfloat16)
    gu = jnp.dot(xb, wgu_s[...], preferred_element_type=jnp.float32)
    gu = gu + bgu_ref[0, 0]
    gate = jnp.minimum(gu, LIMIT)
    glu = gate * jax.nn.sigmoid(gate * ALPHA)
    up1 = jnp.clip(gu, -LIMIT, LIMIT) + 1.0
    up1 = pltpu.roll(up1, 2 * EXPERT_DIM - 1, 1)
    h = (glu * up1).astype(jnp.bfloat16)
    hc = jnp.dot(h, sel_ref[...], preferred_element_type=jnp.float32)
    hc = hc.astype(jnp.bfloat16)
    y = jnp.dot(hc, wd_s[...], preferred_element_type=jnp.float32) + bd_ref[0, 0]
    ys_ref[...] = y


def _tc_ffn(xs, wgu, wd, bgu, bd, block_expert, sel):
    grid_spec = pltpu.PrefetchScalarGridSpec(
        num_scalar_prefetch=1,
        grid=(NBLK,),
        in_specs=[
            pl.BlockSpec((BLK, HIDDEN), lambda b, be: (b, 0)),
            pl.BlockSpec((1, HIDDEN, 2 * EXPERT_DIM), lambda b, be: (be[b], 0, 0)),
            pl.BlockSpec((1, EXPERT_DIM, HIDDEN), lambda b, be: (be[b], 0, 0)),
            pl.BlockSpec((1, 1, 2 * EXPERT_DIM), lambda b, be: (be[b], 0, 0)),
            pl.BlockSpec((1, 1, HIDDEN), lambda b, be: (be[b], 0, 0)),
            pl.BlockSpec((2 * EXPERT_DIM, EXPERT_DIM), lambda b, be: (0, 0)),
        ],
        out_specs=pl.BlockSpec((BLK, HIDDEN), lambda b, be: (b, 0)),
        scratch_shapes=[
            pltpu.VMEM((HIDDEN, 2 * EXPERT_DIM), jnp.bfloat16),
            pltpu.VMEM((EXPERT_DIM, HIDDEN), jnp.bfloat16),
        ],
    )
    return pl.pallas_call(
        _ffn_body,
        grid_spec=grid_spec,
        out_shape=jax.ShapeDtypeStruct((P, HIDDEN), jnp.float32),
    )(block_expert, xs, wgu, wd, bgu, bd, sel)


def kernel(hidden_states, router_indices, routing_weights, gate_up_proj,
           gate_up_proj_bias, down_proj, down_proj_bias):
    batch, seq, hidden = hidden_states.shape
    T = batch * seq
    R = T * TOP_K
    x = hidden_states.reshape(T, hidden).astype(jnp.float32)

    ids = router_indices.reshape(R).astype(jnp.int32)
    oh = (ids[:, None] == jnp.arange(NUM_EXPERTS, dtype=jnp.int32)).astype(jnp.int32)
    counts = jnp.sum(oh, axis=0)
    rank = jnp.take_along_axis(jnp.cumsum(oh, axis=0) - oh,
                               ids[:, None], axis=1)[:, 0]
    nblk_e = (counts + BLK - 1) // BLK
    cum_nblk = jnp.cumsum(nblk_e)
    blk_start = cum_nblk - nblk_e
    dest = blk_start[ids] * BLK + rank
    bid = jnp.arange(NBLK, dtype=jnp.int32)
    block_expert = jnp.minimum(
        jnp.sum((bid[:, None] >= cum_nblk[None, :]).astype(jnp.int32), axis=1),
        NUM_EXPERTS - 1)
    pos = dest.reshape(T, TOP_K).T
    wrow = jnp.take_along_axis(routing_weights, ids.reshape(T, TOP_K),
                               axis=1)
    wb = jnp.broadcast_to(wrow.T[:, :, None], (TOP_K, T, 16))

    bgu = gate_up_proj_bias.reshape(NUM_EXPERTS, 1, 2 * EXPERT_DIM)
    bd = down_proj_bias.reshape(NUM_EXPERTS, 1, HIDDEN)
    r2 = jax.lax.broadcasted_iota(jnp.int32, (2 * EXPERT_DIM, EXPERT_DIM), 0)
    c2 = jax.lax.broadcasted_iota(jnp.int32, (2 * EXPERT_DIM, EXPERT_DIM), 1)
    sel = (r2 == 2 * c2).astype(jnp.bfloat16)

    xs = _sc_dispatch(x, pos)
    ys = _tc_ffn(xs, gate_up_proj, down_proj, bgu, bd,
                 block_expert, sel)
    out = _sc_combine(ys, pos, wb)
    return out.astype(hidden_states.dtype).reshape(batch, seq, hidden)
